# baseline (speedup 1.0000x reference)
import functools

import jax
import jax.numpy as jnp
from jax import lax
from jax.experimental import pallas as pl
from jax.experimental.pallas import tpu as pltpu
from jax.experimental.pallas import tpu_sc as plsc

B = 16384
F = 26
D = 16
NUM_BLOCKS = 5
NC = 2
NS = 16
NW = NC * NS
RPW = B // NW
CR = 128
CE = F * CR
NCHUNK = RPW // CR
G = CR // 16
IB = CE // 128

_THRESH = (100000, 300000, 500000, 700000)


V = 1000000
CW = 2048
NFULL = V // CW
REM = V - NFULL * CW
TAIL = 64
EXTRA = REM - TAIL


def _relayout_body(embt_hbm, tail_hbm, out_hbm, stgp, outb):
    wid = lax.axis_index("s") * NC + lax.axis_index("c")
    lane = lax.iota(jnp.int32, 16)

    def do_chunk(c0, cw):
        pltpu.sync_copy(embt_hbm.at[:, pl.ds(c0, cw)],
                        stgp.at[:, pl.ds(0, cw)])

        def col(i, _):
            base = i * 8
            for u in range(8):
                v = plsc.load_gather(
                    stgp, [lane, jnp.full((16,), u, jnp.int32) + base])
                outb[pl.ds((base + u) * 16, 16)] = v
            return 0
        lax.fori_loop(0, cw // 8, col, 0)
        pltpu.sync_copy(outb.at[pl.ds(0, cw * 16)],
                        out_hbm.at[pl.ds(c0 * 16, cw * 16)])

    def t_body(t, _):
        cid = wid + t * NW

        @pl.when(cid < NFULL)
        def _():
            do_chunk(pl.multiple_of(cid * CW, CW), CW)

        @pl.when(cid == NFULL)
        def _():
            do_chunk(pl.multiple_of(NFULL * CW, 512), EXTRA)
        return 0
    lax.fori_loop(0, (NFULL + NW) // NW, t_body, 0)

    @pl.when(wid == 0)
    def _():
        pltpu.sync_copy(tail_hbm, out_hbm.at[pl.ds((V - TAIL) * 16, TAIL * 16)])


@jax.jit
def _relayout_sc(embt, tail_lin):
    mesh = plsc.VectorSubcoreMesh(core_axis_name="c", subcore_axis_name="s")
    return pl.kernel(
        _relayout_body,
        out_type=jax.ShapeDtypeStruct((V * D,), jnp.float32),
        mesh=mesh,
        compiler_params=pltpu.CompilerParams(
            use_tc_tiling_on_sc=True, needs_layout_passes=False),
        scratch_types=[
            pltpu.VMEM((16, 2049), jnp.float32),
            pltpu.VMEM((CW * 16,), jnp.float32),
        ],
    )(embt, tail_lin)


def _fm_body(ids_hbm, vals_hbm, emb_hbm, fm_hbm, mask_hbm, bias_hbm,
             out_hbm, ids_v, vals_v, emb_v, fmv_v, mids_v, prods_v,
             mask_v, bias_v, out_v, sem_e, sem_f):
    wid = lax.axis_index("s") * NC + lax.axis_index("c")
    base_elem = wid * (RPW * F)

    pltpu.sync_copy(mask_hbm, mask_v)
    pltpu.sync_copy(bias_hbm, bias_v)
    bias_vec = bias_v[...]
    zeros = jnp.zeros((16,), jnp.float32)
    lane = lax.iota(jnp.int32, 16)
    rot8 = (lane + 8) % 16
    rot4 = (lane + 4) % 16
    rot2 = (lane + 2) % 16
    rot1 = (lane + 1) % 16

    def _rotsum(v):
        for r in (rot8, rot4, rot2, rot1):
            v = v + v.at[r].get(mode="promise_in_bounds")
        return v

    def chunk_body(c, _):
        e0 = pl.multiple_of(base_elem + c * CE, CE)
        pltpu.sync_copy(ids_hbm.at[pl.ds(e0, CE)], ids_v)
        pltpu.sync_copy(vals_hbm.at[pl.ds(e0, CE)], vals_v)

        for j in range(IB):
            pltpu.make_async_copy(
                emb_hbm.at[ids_v.at[pl.ds(j * 128, 128)]],
                emb_v.at[pl.ds(j * 128, 128)], sem_e).start()
            pltpu.make_async_copy(
                fm_hbm.at[ids_v.at[pl.ds(j * 128, 128)]],
                fmv_v.at[pl.ds(j * 128, 128)], sem_f).start()
        for j in range(IB):
            pltpu.make_async_copy(
                emb_hbm.at[ids_v.at[pl.ds(0, 128)]],
                emb_v.at[pl.ds(0, 128)], sem_e).wait()
            pltpu.make_async_copy(
                fm_hbm.at[ids_v.at[pl.ds(0, 128)]],
                fmv_v.at[pl.ds(0, 128)], sem_f).wait()

        def pre_body(k, _):
            sl = pl.ds(k * 16, 16)
            id_vec = ids_v[sl]
            mid = ((id_vec >= _THRESH[0]).astype(jnp.int32)
                   + (id_vec >= _THRESH[1]).astype(jnp.int32)
                   + (id_vec >= _THRESH[2]).astype(jnp.int32)
                   + (id_vec >= _THRESH[3]).astype(jnp.int32))
            mids_v[sl] = mid
            prods_v[sl] = vals_v[sl] * fmv_v[sl]
            return 0
        lax.fori_loop(0, CE // 16, pre_body, 0)

        def group_body(g, _):
            gbase = g * (16 * F)
            lin = bias_vec
            for f in range(F):
                lin = lin + plsc.load_gather(prods_v, [gbase + f + lane * F])

            def row_body(r, x_acc):
                ebase = gbase + r * F
                vrow0 = vals_v[pl.ds(ebase, 16)]
                vrow1 = vals_v[pl.ds(ebase + F - 16, 16)]
                mrow0 = mids_v[pl.ds(ebase, 16)]
                mrow1 = mids_v[pl.ds(ebase + F - 16, 16)]
                acc = zeros
                accsq = zeros
                for f in range(F):
                    if f < 16:
                        va = vrow0[f]
                        mi = mrow0[f]
                    else:
                        va = vrow1[f - (F - 16)]
                        mi = mrow1[f - (F - 16)]
                    row = emb_v[ebase + f, :]
                    mrow = mask_v[mi, :]
                    tv = row * mrow * va
                    acc = acc + tv
                    accsq = accsq + tv * tv
                w = acc * acc - accsq
                x = _rotsum(w)
                return jnp.where(lane == r, x, x_acc)

            x_vec = lax.fori_loop(0, 16, row_body, zeros)
            x = lin + 0.5 * x_vec
            out_v[pl.ds(c * CR + g * 16, 16)] = 1.0 / (1.0 + jnp.exp(-x))
            return 0
        lax.fori_loop(0, G, group_body, 0)
        return 0

    lax.fori_loop(0, NCHUNK, chunk_body, 0)
    pltpu.sync_copy(out_v, out_hbm.at[pl.ds(wid * RPW, RPW)])


@jax.jit
def _fm_sc(ids_flat, vals_flat, emb_table, fm_flat, mask_table, bias_vec):
    mesh = plsc.VectorSubcoreMesh(core_axis_name="c", subcore_axis_name="s")
    return pl.kernel(
        _fm_body,
        out_type=jax.ShapeDtypeStruct((B,), jnp.float32),
        mesh=mesh,
        compiler_params=pltpu.CompilerParams(
            use_tc_tiling_on_sc=False, needs_layout_passes=False),
        scratch_types=[
            pltpu.VMEM((CE,), jnp.int32),
            pltpu.VMEM((CE,), jnp.float32),
            pltpu.VMEM((CE, D), jnp.float32),
            pltpu.VMEM((CE,), jnp.float32),
            pltpu.VMEM((CE,), jnp.int32),
            pltpu.VMEM((CE,), jnp.float32),
            pltpu.VMEM((NUM_BLOCKS, 16), jnp.float32),
            pltpu.VMEM((16,), jnp.float32),
            pltpu.VMEM((RPW,), jnp.float32),
            pltpu.SemaphoreType.DMA,
            pltpu.SemaphoreType.DMA,
        ],
    )(ids_flat, vals_flat, emb_table, fm_flat, mask_table, bias_vec)


def kernel(feature_ids, feature_vals, emb_table, alpha, fm_w, fm_bias):
    tail_lin = lax.optimization_barrier(
        emb_table[V - TAIL:, :].reshape(-1))
    emb_lin = _relayout_sc(emb_table.T, tail_lin).reshape(emb_table.shape)
    ids_flat = lax.optimization_barrier(
        feature_ids.reshape(-1).astype(jnp.int32))
    vals_flat = lax.optimization_barrier(feature_vals.reshape(-1))
    fm_flat = lax.optimization_barrier(fm_w.reshape(-1))
    s = jnp.arange(NUM_BLOCKS, dtype=jnp.float32)
    abw = jnp.clip(alpha[None, :] - s[:, None], 0.0, 1.0)
    mask = jnp.repeat(abw, D // abw.shape[1], axis=1)
    msum = mask.sum(axis=1, keepdims=True)
    mask_n = mask / (msum + 1e-6) * 4.0
    bias_vec = jnp.full((16,), fm_bias, dtype=jnp.float32)
    return _fm_sc(ids_flat, vals_flat, emb_lin, fm_flat, mask_n, bias_vec)


# speedup vs baseline: 1.0809x; 1.0809x over previous
import functools

import jax
import jax.numpy as jnp
from jax import lax
from jax.experimental import pallas as pl
from jax.experimental.pallas import tpu as pltpu
from jax.experimental.pallas import tpu_sc as plsc

B = 16384
F = 26
D = 16
NUM_BLOCKS = 5
NC = 2
NS = 16
NW = NC * NS
RPW = B // NW
CR = 128
CE = F * CR
NCHUNK = RPW // CR
G = CR // 16
IB = CE // 128

_THRESH = (100000, 300000, 500000, 700000)


V = 1000000
CW = 1536
NFULL = V // CW
TAIL = 64
NCH_HI = -(-NFULL // NW)
NCH_EXTRA = NFULL - (NCH_HI - 1) * NW


def _relayout_body(embt_hbm, tail_hbm, out_hbm, stg_a, stg_b, outb,
                   sem_a, sem_b):
    wid = lax.axis_index("s") * NC + lax.axis_index("c")
    lane = lax.iota(jnp.int32, 16)
    nch = jnp.where(wid < NCH_EXTRA, NCH_HI, NCH_HI - 1)

    def c0_of(t):
        return pl.multiple_of((wid + t * NW) * CW, 128)

    def fire_in(t, stg, sem):
        pltpu.make_async_copy(
            embt_hbm.at[:, pl.ds(c0_of(t), CW)],
            stg.at[:, pl.ds(0, CW)], sem).start()

    def wait_in(stg, sem):
        pltpu.make_async_copy(
            embt_hbm.at[:, pl.ds(0, CW)],
            stg.at[:, pl.ds(0, CW)], sem).wait()

    def do_chunk(t, stg):
        def col(i, _):
            base = i * 8
            for u in range(8):
                v = plsc.load_gather(
                    stg, [lane, jnp.full((16,), u, jnp.int32) + base])
                outb[pl.ds((base + u) * 16, 16)] = v
            return 0
        lax.fori_loop(0, CW // 8, col, 0)
        pltpu.sync_copy(outb, out_hbm.at[pl.ds(c0_of(t) * 16, CW * 16)])

    fire_in(0, stg_a, sem_a)

    def body2(t2, _):
        ta = 2 * t2
        tb = ta + 1

        @pl.when(ta < nch)
        def _():
            @pl.when(tb < nch)
            def _():
                fire_in(tb, stg_b, sem_b)
            wait_in(stg_a, sem_a)
            do_chunk(ta, stg_a)

            @pl.when(tb < nch)
            def _():
                @pl.when(tb + 1 < nch)
                def _():
                    fire_in(tb + 1, stg_a, sem_a)
                wait_in(stg_b, sem_b)
                do_chunk(tb, stg_b)
        return 0
    lax.fori_loop(0, (NCH_HI + 1) // 2, body2, 0)

    @pl.when(wid == 0)
    def _():
        pltpu.sync_copy(tail_hbm, out_hbm.at[pl.ds((V - TAIL) * 16, TAIL * 16)])


@jax.jit
def _relayout_sc(embt, tail_lin):
    mesh = plsc.VectorSubcoreMesh(core_axis_name="c", subcore_axis_name="s")
    return pl.kernel(
        _relayout_body,
        out_type=jax.ShapeDtypeStruct((V * D,), jnp.float32),
        mesh=mesh,
        compiler_params=pltpu.CompilerParams(
            use_tc_tiling_on_sc=True, needs_layout_passes=False),
        scratch_types=[
            pltpu.VMEM((16, CW + 1), jnp.float32),
            pltpu.VMEM((16, CW + 1), jnp.float32),
            pltpu.VMEM((CW * 16,), jnp.float32),
            pltpu.SemaphoreType.DMA,
            pltpu.SemaphoreType.DMA,
        ],
    )(embt, tail_lin)


def _fm_body(ids_hbm, vals_hbm, emb_hbm, fm_hbm, mask_hbm, bias_hbm,
             out_hbm, ids_v, vals_v, emb_v, fmv_v, mids_v, prods_v,
             mask_v, bias_v, out_v, sem_e, sem_f):
    wid = lax.axis_index("s") * NC + lax.axis_index("c")
    base_elem = wid * (RPW * F)

    pltpu.sync_copy(mask_hbm, mask_v)
    pltpu.sync_copy(bias_hbm, bias_v)
    bias_vec = bias_v[...]
    zeros = jnp.zeros((16,), jnp.float32)
    lane = lax.iota(jnp.int32, 16)
    rot8 = (lane + 8) % 16
    rot4 = (lane + 4) % 16
    rot2 = (lane + 2) % 16
    rot1 = (lane + 1) % 16

    def _rotsum(v):
        for r in (rot8, rot4, rot2, rot1):
            v = v + v.at[r].get(mode="promise_in_bounds")
        return v

    def chunk_body(c, _):
        e0 = pl.multiple_of(base_elem + c * CE, CE)
        pltpu.sync_copy(ids_hbm.at[pl.ds(e0, CE)], ids_v)
        pltpu.sync_copy(vals_hbm.at[pl.ds(e0, CE)], vals_v)

        for j in range(IB):
            pltpu.make_async_copy(
                emb_hbm.at[ids_v.at[pl.ds(j * 128, 128)]],
                emb_v.at[pl.ds(j * 128, 128)], sem_e).start()
            pltpu.make_async_copy(
                fm_hbm.at[ids_v.at[pl.ds(j * 128, 128)]],
                fmv_v.at[pl.ds(j * 128, 128)], sem_f).start()
        for j in range(IB):
            pltpu.make_async_copy(
                emb_hbm.at[ids_v.at[pl.ds(0, 128)]],
                emb_v.at[pl.ds(0, 128)], sem_e).wait()
            pltpu.make_async_copy(
                fm_hbm.at[ids_v.at[pl.ds(0, 128)]],
                fmv_v.at[pl.ds(0, 128)], sem_f).wait()

        def pre_body(k, _):
            sl = pl.ds(k * 16, 16)
            id_vec = ids_v[sl]
            mid = ((id_vec >= _THRESH[0]).astype(jnp.int32)
                   + (id_vec >= _THRESH[1]).astype(jnp.int32)
                   + (id_vec >= _THRESH[2]).astype(jnp.int32)
                   + (id_vec >= _THRESH[3]).astype(jnp.int32))
            mids_v[sl] = mid
            prods_v[sl] = vals_v[sl] * fmv_v[sl]
            return 0
        lax.fori_loop(0, CE // 16, pre_body, 0)

        def group_body(g, _):
            gbase = g * (16 * F)
            lin = bias_vec
            for f in range(F):
                lin = lin + plsc.load_gather(prods_v, [gbase + f + lane * F])

            def row_body(r, x_acc):
                ebase = gbase + r * F
                vrow0 = vals_v[pl.ds(ebase, 16)]
                vrow1 = vals_v[pl.ds(ebase + F - 16, 16)]
                mrow0 = mids_v[pl.ds(ebase, 16)]
                mrow1 = mids_v[pl.ds(ebase + F - 16, 16)]
                acc = zeros
                accsq = zeros
                for f in range(F):
                    if f < 16:
                        va = vrow0[f]
                        mi = mrow0[f]
                    else:
                        va = vrow1[f - (F - 16)]
                        mi = mrow1[f - (F - 16)]
                    row = emb_v[ebase + f, :]
                    mrow = mask_v[mi, :]
                    tv = row * mrow * va
                    acc = acc + tv
                    accsq = accsq + tv * tv
                w = acc * acc - accsq
                x = _rotsum(w)
                return jnp.where(lane == r, x, x_acc)

            x_vec = lax.fori_loop(0, 16, row_body, zeros)
            x = lin + 0.5 * x_vec
            out_v[pl.ds(c * CR + g * 16, 16)] = 1.0 / (1.0 + jnp.exp(-x))
            return 0
        lax.fori_loop(0, G, group_body, 0)
        return 0

    lax.fori_loop(0, NCHUNK, chunk_body, 0)
    pltpu.sync_copy(out_v, out_hbm.at[pl.ds(wid * RPW, RPW)])


@jax.jit
def _fm_sc(ids_flat, vals_flat, emb_table, fm_flat, mask_table, bias_vec):
    mesh = plsc.VectorSubcoreMesh(core_axis_name="c", subcore_axis_name="s")
    return pl.kernel(
        _fm_body,
        out_type=jax.ShapeDtypeStruct((B,), jnp.float32),
        mesh=mesh,
        compiler_params=pltpu.CompilerParams(
            use_tc_tiling_on_sc=False, needs_layout_passes=False),
        scratch_types=[
            pltpu.VMEM((CE,), jnp.int32),
            pltpu.VMEM((CE,), jnp.float32),
            pltpu.VMEM((CE, D), jnp.float32),
            pltpu.VMEM((CE,), jnp.float32),
            pltpu.VMEM((CE,), jnp.int32),
            pltpu.VMEM((CE,), jnp.float32),
            pltpu.VMEM((NUM_BLOCKS, 16), jnp.float32),
            pltpu.VMEM((16,), jnp.float32),
            pltpu.VMEM((RPW,), jnp.float32),
            pltpu.SemaphoreType.DMA,
            pltpu.SemaphoreType.DMA,
        ],
    )(ids_flat, vals_flat, emb_table, fm_flat, mask_table, bias_vec)


def kernel(feature_ids, feature_vals, emb_table, alpha, fm_w, fm_bias):
    tail_lin = lax.optimization_barrier(
        emb_table[V - TAIL:, :].reshape(-1))
    emb_lin = _relayout_sc(emb_table.T, tail_lin).reshape(emb_table.shape)
    ids_flat = lax.optimization_barrier(
        feature_ids.reshape(-1).astype(jnp.int32))
    vals_flat = lax.optimization_barrier(feature_vals.reshape(-1))
    fm_flat = lax.optimization_barrier(fm_w.reshape(-1))
    s = jnp.arange(NUM_BLOCKS, dtype=jnp.float32)
    abw = jnp.clip(alpha[None, :] - s[:, None], 0.0, 1.0)
    mask = jnp.repeat(abw, D // abw.shape[1], axis=1)
    msum = mask.sum(axis=1, keepdims=True)
    mask_n = mask / (msum + 1e-6) * 4.0
    bias_vec = jnp.full((16,), fm_bias, dtype=jnp.float32)
    return _fm_sc(ids_flat, vals_flat, emb_lin, fm_flat, mask_n, bias_vec)
